# baseline (speedup 1.0000x reference)
import functools

import jax
import jax.numpy as jnp
from jax import lax
from jax.experimental import pallas as pl
from jax.experimental.pallas import tpu as pltpu
from jax.experimental.pallas import tpu_sc as plsc

_N = 10000
_E = 320000
_D = 128
_B = 64
_K = 30
_CONV_OUT = 32
_CONV_W = 3
_NUM_CLASSES = 10

_ROWS_PER_BLK = 2000

_EC = 128
_NT = 16
_NW = 32
_NP = 10240
_ROWS_T = _NP // _NT
_NCH_T = 80
_EPAD = _NW * _NCH_T * _EC - _E


def _sage_dense_body(agg0_ref, agg1_ref, cw0_ref, cw1_ref, x_ref, wl_ref,
                     bl_ref, wr_ref, o_ref):
    cnt = cw0_ref[...][:, 0:1] + cw1_ref[...][:, 0:1]
    inv = 1.0 / jnp.maximum(cnt, 1.0)
    m = (agg0_ref[...] + agg1_ref[...]) * inv
    acc = jax.lax.dot_general(m, wl_ref[...], (((1,), (1,)), ((), ())),
                              preferred_element_type=jnp.float32)
    acc = acc + jax.lax.dot_general(x_ref[...], wr_ref[...],
                                    (((1,), (1,)), ((), ())),
                                    preferred_element_type=jnp.float32)
    acc = acc + bl_ref[...]
    o_ref[...] = jnp.maximum(acc, 0.0)


def _sage_dense(agg0, agg1, cw0, cw1, x, Wl, bl, Wr):
    nblk = _N // _ROWS_PER_BLK
    return pl.pallas_call(
        _sage_dense_body,
        grid=(nblk,),
        in_specs=[
            pl.BlockSpec((_ROWS_PER_BLK, _D), lambda i: (i, 0)),
            pl.BlockSpec((_ROWS_PER_BLK, _D), lambda i: (i, 0)),
            pl.BlockSpec((_ROWS_PER_BLK, _D), lambda i: (i, 0)),
            pl.BlockSpec((_ROWS_PER_BLK, _D), lambda i: (i, 0)),
            pl.BlockSpec((_ROWS_PER_BLK, _D), lambda i: (i, 0)),
            pl.BlockSpec((_D, _D), lambda i: (0, 0)),
            pl.BlockSpec((1, _D), lambda i: (0, 0)),
            pl.BlockSpec((_D, _D), lambda i: (0, 0)),
        ],
        out_specs=pl.BlockSpec((_ROWS_PER_BLK, _D), lambda i: (i, 0)),
        out_shape=jax.ShapeDtypeStruct((_N, _D), jnp.float32),
    )(agg0, agg1, cw0, cw1, x, Wl, bl.reshape(1, _D), Wr)


def _head_body(p_ref, wc_ref, bc_ref, w1_ref, b1_ref, w2_ref, b2_ref, o_ref):
    p = p_ref[...].reshape(_B * _K, _D)
    q = jax.lax.dot_general(p, wc_ref[...], (((1,), (0,)), ((), ())),
                            preferred_element_type=jnp.float32)
    q = q.reshape(_B, _K, _CONV_W * _CONV_OUT)
    t_out = _K - _CONV_W + 1
    y = (q[:, 0:t_out, 0:_CONV_OUT]
         + q[:, 1:t_out + 1, _CONV_OUT:2 * _CONV_OUT]
         + q[:, 2:t_out + 2, 2 * _CONV_OUT:3 * _CONV_OUT])
    y = jnp.maximum(y + bc_ref[...], 0.0)
    z = y.reshape(_B, t_out * _CONV_OUT)
    z = jax.lax.dot_general(z, w1_ref[...], (((1,), (0,)), ((), ())),
                            preferred_element_type=jnp.float32)
    z = jnp.maximum(z + b1_ref[...], 0.0)
    logits = jax.lax.dot_general(z, w2_ref[...], (((1,), (0,)), ((), ())),
                                 preferred_element_type=jnp.float32)
    logits = logits + b2_ref[...]
    mx = jnp.max(logits, axis=-1, keepdims=True)
    s = logits - mx
    lse = jnp.log(jnp.sum(jnp.exp(s), axis=-1, keepdims=True))
    o_ref[...] = s - lse


def _head(p, Wc, bc, W1, b1, W2, b2):
    t_out = _K - _CONV_W + 1
    wc_all = jnp.concatenate([Wc[:, :, dt].T for dt in range(_CONV_W)], axis=1)
    w1p = W1.reshape(_D, _CONV_OUT, t_out).transpose(2, 1, 0).reshape(t_out * _CONV_OUT, _D)
    return pl.pallas_call(
        _head_body,
        out_shape=jax.ShapeDtypeStruct((_B, _NUM_CLASSES), jnp.float32),
    )(p, wc_all, bc.reshape(1, 1, _CONV_OUT), w1p, b1.reshape(1, _D),
      W2.T, b2.reshape(1, _NUM_CLASSES))


def _make_sc_agg(with_cnt):
    outs = [jax.ShapeDtypeStruct((_NP, _D), jnp.float32),
            jax.ShapeDtypeStruct((_NP, _D), jnp.float32)]
    scratch = [
        pltpu.VMEM((_EC,), jnp.int32),
        pltpu.VMEM((_EC,), jnp.int32),
        pltpu.VMEM((_EC,), jnp.int32),
        pltpu.VMEM((_EC,), jnp.int32),
        pltpu.VMEM((_EC, _D), jnp.float32),
        pltpu.VMEM((_EC, _D), jnp.float32),
        pltpu.VMEM_SHARED((_NP, _D), jnp.float32),
        pltpu.SemaphoreType.DMA,
        pltpu.SemaphoreType.DMA,
        pltpu.SemaphoreType.DMA,
        pltpu.SemaphoreType.DMA,
    ]
    if with_cnt:
        outs += [jax.ShapeDtypeStruct((_NP, _D), jnp.float32),
                 jax.ShapeDtypeStruct((_NP, _D), jnp.float32)]
    mesh = plsc.VectorSubcoreMesh(core_axis_name="c", subcore_axis_name="s")

    def body(x, srcp, dstp, z64, *refs):
        if with_cnt:
            (ones, agg0, agg1, cntw0, cntw1, sbuf0, sbuf1, dbuf0, dbuf1,
             rows0, rows1, aggsp, gsem0, gsem1, asem0, asem1) = refs
        else:
            (agg0, agg1, sbuf0, sbuf1, dbuf0, dbuf1,
             rows0, rows1, aggsp, gsem0, gsem1, asem0, asem1) = refs
        core = lax.axis_index("c")
        sid = lax.axis_index("s")
        wid = core * _NT + sid
        r0 = sid * _ROWS_T

        pltpu.sync_copy(z64.at[pl.ds(0, _EC)], rows0)
        for b in range(_ROWS_T // _EC):
            pltpu.sync_copy(rows0, aggsp.at[pl.ds(r0 + b * _EC, _EC)])
        plsc.subcore_barrier()

        e_base = wid * _NCH_T * _EC

        def ldidx(c, sbuf, dbuf):
            off = pl.multiple_of(e_base + c * _EC, _EC)
            pltpu.sync_copy(srcp.at[pl.ds(off, _EC)], sbuf)
            pltpu.sync_copy(dstp.at[pl.ds(off, _EC)], dbuf)

        def gath(buf, idx, sem):
            pltpu.async_copy(x.at[idx], buf, sem)

        def gath_wait(buf, idx, sem):
            pltpu.make_async_copy(x.at[idx], buf, sem).wait()

        def sadd(buf, idx, sem):
            pltpu.async_copy(buf, aggsp.at[idx], sem, add=True)

        def sadd_wait(buf, idx, sem):
            pltpu.make_async_copy(buf, aggsp.at[idx], sem).wait()

        ldidx(0, sbuf0, dbuf0)
        gath(rows0, sbuf0, gsem0)
        ldidx(1, sbuf1, dbuf1)
        gath(rows1, sbuf1, gsem1)

        def pipe(i, carry):
            c = 2 * i
            gath_wait(rows0, sbuf0, gsem0)
            sadd(rows0, dbuf0, asem0)
            gath_wait(rows1, sbuf1, gsem1)
            sadd(rows1, dbuf1, asem1)
            sadd_wait(rows0, dbuf0, asem0)
            ldidx(c + 2, sbuf0, dbuf0)
            gath(rows0, sbuf0, gsem0)
            sadd_wait(rows1, dbuf1, asem1)
            ldidx(c + 3, sbuf1, dbuf1)
            gath(rows1, sbuf1, gsem1)
            return carry

        lax.fori_loop(0, _NCH_T // 2 - 1, pipe, 0)
        gath_wait(rows0, sbuf0, gsem0)
        sadd(rows0, dbuf0, asem0)
        gath_wait(rows1, sbuf1, gsem1)
        sadd(rows1, dbuf1, asem1)
        sadd_wait(rows0, dbuf0, asem0)
        sadd_wait(rows1, dbuf1, asem1)
        plsc.subcore_barrier()

        for b in range(_ROWS_T // _EC):
            pltpu.sync_copy(aggsp.at[pl.ds(r0 + b * _EC, _EC)], rows0)

            @pl.when(core == 0)
            def _():
                pltpu.sync_copy(rows0, agg0.at[pl.ds(r0 + b * _EC, _EC)])

            @pl.when(core == 1)
            def _():
                pltpu.sync_copy(rows0, agg1.at[pl.ds(r0 + b * _EC, _EC)])

        if with_cnt:
            pltpu.sync_copy(z64.at[pl.ds(0, _EC)], rows1)
            for b in range(_ROWS_T // _EC):
                pltpu.sync_copy(rows1, aggsp.at[pl.ds(r0 + b * _EC, _EC)])
            pltpu.sync_copy(ones, rows0)
            plsc.subcore_barrier()

            ldidx(0, sbuf0, dbuf0)
            sadd(rows0, dbuf0, asem0)
            ldidx(1, sbuf1, dbuf1)
            sadd(rows0, dbuf1, asem1)

            def pipe_cnt(i, carry):
                c = 2 * i
                sadd_wait(rows0, dbuf0, asem0)
                ldidx(c + 2, sbuf0, dbuf0)
                sadd(rows0, dbuf0, asem0)
                sadd_wait(rows0, dbuf1, asem1)
                ldidx(c + 3, sbuf1, dbuf1)
                sadd(rows0, dbuf1, asem1)
                return carry

            lax.fori_loop(0, _NCH_T // 2 - 1, pipe_cnt, 0)
            sadd_wait(rows0, dbuf0, asem0)
            sadd_wait(rows0, dbuf1, asem1)
            plsc.subcore_barrier()

            for b in range(_ROWS_T // _EC):
                pltpu.sync_copy(aggsp.at[pl.ds(r0 + b * _EC, _EC)], rows0)

                @pl.when(core == 0)
                def _():
                    pltpu.sync_copy(rows0, cntw0.at[pl.ds(r0 + b * _EC, _EC)])

                @pl.when(core == 1)
                def _():
                    pltpu.sync_copy(rows0, cntw1.at[pl.ds(r0 + b * _EC, _EC)])

    return pl.kernel(body, out_type=tuple(outs), mesh=mesh,
                     scratch_types=scratch)


def _sc_agg(x, srcp, dstp, zeros64, with_cnt, ones=None):
    k = _make_sc_agg(with_cnt)
    if with_cnt:
        return k(x, srcp, dstp, zeros64, ones)
    return k(x, srcp, dstp, zeros64)


def _sort_pool_xla(x, batch):
    score = x[:, -1]
    order = jnp.lexsort((-score, batch))
    bs = batch[order]
    xs = x[order]
    counts = jnp.bincount(batch, length=_B)
    starts = jnp.cumsum(counts) - counts
    pos = jnp.arange(x.shape[0]) - starts[bs]
    mask = pos < _K
    safe = jnp.where(mask, pos, 0)
    vals = jnp.where(mask[:, None], xs, 0.0)
    return jnp.zeros((_B, _K, x.shape[1]), x.dtype).at[bs, safe].add(vals)


def kernel(x, edge_index, batch, k, Wl1, bl1, Wr1, Wl2, bl2, Wr2, Wl3, bl3,
           Wr3, Wc, bc, W1, b1, W2, b2):
    src, dst = edge_index[0], edge_index[1]
    srcp = jnp.concatenate([src, jnp.zeros((_EPAD,), jnp.int32)])
    dstp = jnp.concatenate([dst, jnp.full((_EPAD,), _N, jnp.int32)])
    zeros64 = jnp.zeros((_NP, _D), jnp.float32)
    ones = jnp.full((_EC, _D), 1.0, jnp.float32)

    agg0, agg1, cw0, cw1 = _sc_agg(x, srcp, dstp, zeros64, True, ones)
    h = _sage_dense(agg0, agg1, cw0, cw1, x, Wl1, bl1, Wr1)
    agg0, agg1 = _sc_agg(h, srcp, dstp, zeros64, False)
    h = _sage_dense(agg0, agg1, cw0, cw1, h, Wl2, bl2, Wr2)
    agg0, agg1 = _sc_agg(h, srcp, dstp, zeros64, False)
    h = _sage_dense(agg0, agg1, cw0, cw1, h, Wl3, bl3, Wr3)
    p = _sort_pool_xla(h, batch)
    return _head(p, Wc, bc, W1, b1, W2, b2)


# speedup vs baseline: 1.0009x; 1.0009x over previous
import functools

import jax
import jax.numpy as jnp
from jax import lax
from jax.experimental import pallas as pl
from jax.experimental.pallas import tpu as pltpu
from jax.experimental.pallas import tpu_sc as plsc

_N = 10000
_E = 320000
_D = 128
_B = 64
_K = 30
_CONV_OUT = 32
_CONV_W = 3
_NUM_CLASSES = 10

_ROWS_PER_BLK = 2000

_EC = 128
_NT = 16
_NW = 32
_NP = 10240
_ROWS_T = _NP // _NT
_NCH_T = 80
_EPAD = _NW * _NCH_T * _EC - _E


def _sage_dense_body(agg0_ref, agg1_ref, cw0_ref, cw1_ref, x_ref, wl_ref,
                     bl_ref, wr_ref, o_ref):
    cnt = cw0_ref[...][:, 0:1] + cw1_ref[...][:, 0:1]
    inv = 1.0 / jnp.maximum(cnt, 1.0)
    m = (agg0_ref[...] + agg1_ref[...]) * inv
    acc = jax.lax.dot_general(m, wl_ref[...], (((1,), (1,)), ((), ())),
                              preferred_element_type=jnp.float32)
    acc = acc + jax.lax.dot_general(x_ref[...], wr_ref[...],
                                    (((1,), (1,)), ((), ())),
                                    preferred_element_type=jnp.float32)
    acc = acc + bl_ref[...]
    o_ref[...] = jnp.maximum(acc, 0.0)


def _sage_dense(agg0, agg1, cw0, cw1, x, Wl, bl, Wr):
    nblk = _N // _ROWS_PER_BLK
    return pl.pallas_call(
        _sage_dense_body,
        grid=(nblk,),
        in_specs=[
            pl.BlockSpec((_ROWS_PER_BLK, _D), lambda i: (i, 0)),
            pl.BlockSpec((_ROWS_PER_BLK, _D), lambda i: (i, 0)),
            pl.BlockSpec((_ROWS_PER_BLK, _D), lambda i: (i, 0)),
            pl.BlockSpec((_ROWS_PER_BLK, _D), lambda i: (i, 0)),
            pl.BlockSpec((_ROWS_PER_BLK, _D), lambda i: (i, 0)),
            pl.BlockSpec((_D, _D), lambda i: (0, 0)),
            pl.BlockSpec((1, _D), lambda i: (0, 0)),
            pl.BlockSpec((_D, _D), lambda i: (0, 0)),
        ],
        out_specs=pl.BlockSpec((_ROWS_PER_BLK, _D), lambda i: (i, 0)),
        out_shape=jax.ShapeDtypeStruct((_N, _D), jnp.float32),
    )(agg0, agg1, cw0, cw1, x, Wl, bl.reshape(1, _D), Wr)


def _head_body(p_ref, wc_ref, bc_ref, w1_ref, b1_ref, w2_ref, b2_ref, o_ref):
    p = p_ref[...].reshape(_B * _K, _D)
    q = jax.lax.dot_general(p, wc_ref[...], (((1,), (0,)), ((), ())),
                            preferred_element_type=jnp.float32)
    q = q.reshape(_B, _K, _CONV_W * _CONV_OUT)
    t_out = _K - _CONV_W + 1
    y = (q[:, 0:t_out, 0:_CONV_OUT]
         + q[:, 1:t_out + 1, _CONV_OUT:2 * _CONV_OUT]
         + q[:, 2:t_out + 2, 2 * _CONV_OUT:3 * _CONV_OUT])
    y = jnp.maximum(y + bc_ref[...], 0.0)
    z = y.reshape(_B, t_out * _CONV_OUT)
    z = jax.lax.dot_general(z, w1_ref[...], (((1,), (0,)), ((), ())),
                            preferred_element_type=jnp.float32)
    z = jnp.maximum(z + b1_ref[...], 0.0)
    logits = jax.lax.dot_general(z, w2_ref[...], (((1,), (0,)), ((), ())),
                                 preferred_element_type=jnp.float32)
    logits = logits + b2_ref[...]
    mx = jnp.max(logits, axis=-1, keepdims=True)
    s = logits - mx
    lse = jnp.log(jnp.sum(jnp.exp(s), axis=-1, keepdims=True))
    o_ref[...] = s - lse


def _head(p, Wc, bc, W1, b1, W2, b2):
    t_out = _K - _CONV_W + 1
    wc_all = jnp.concatenate([Wc[:, :, dt].T for dt in range(_CONV_W)], axis=1)
    w1p = W1.reshape(_D, _CONV_OUT, t_out).transpose(2, 1, 0).reshape(t_out * _CONV_OUT, _D)
    return pl.pallas_call(
        _head_body,
        out_shape=jax.ShapeDtypeStruct((_B, _NUM_CLASSES), jnp.float32),
    )(p, wc_all, bc.reshape(1, 1, _CONV_OUT), w1p, b1.reshape(1, _D),
      W2.T, b2.reshape(1, _NUM_CLASSES))


def _make_sc_agg(with_cnt):
    outs = [jax.ShapeDtypeStruct((_NP, _D), jnp.float32),
            jax.ShapeDtypeStruct((_NP, _D), jnp.float32)]
    scratch = [
        pltpu.VMEM((_EC,), jnp.int32),
        pltpu.VMEM((_EC,), jnp.int32),
        pltpu.VMEM((_EC,), jnp.int32),
        pltpu.VMEM((_EC,), jnp.int32),
        pltpu.VMEM((_EC, _D), jnp.float32),
        pltpu.VMEM((_EC, _D), jnp.float32),
        pltpu.VMEM_SHARED((_NP, _D), jnp.float32),
        pltpu.SemaphoreType.DMA,
        pltpu.SemaphoreType.DMA,
        pltpu.SemaphoreType.DMA,
        pltpu.SemaphoreType.DMA,
    ]
    if with_cnt:
        outs += [jax.ShapeDtypeStruct((_NP, _D), jnp.float32),
                 jax.ShapeDtypeStruct((_NP, _D), jnp.float32)]
    mesh = plsc.VectorSubcoreMesh(core_axis_name="c", subcore_axis_name="s")

    def body(x, srcp, dstp, z64, *refs):
        if with_cnt:
            (ones, agg0, agg1, cntw0, cntw1, sbuf0, sbuf1, dbuf0, dbuf1,
             rows0, rows1, aggsp, gsem0, gsem1, asem0, asem1) = refs
        else:
            (agg0, agg1, sbuf0, sbuf1, dbuf0, dbuf1,
             rows0, rows1, aggsp, gsem0, gsem1, asem0, asem1) = refs
        core = lax.axis_index("c")
        sid = lax.axis_index("s")
        wid = core * _NT + sid
        r0 = sid * _ROWS_T

        pltpu.sync_copy(z64.at[pl.ds(0, _EC)], rows0)
        for b in range(_ROWS_T // _EC):
            pltpu.sync_copy(rows0, aggsp.at[pl.ds(r0 + b * _EC, _EC)])
        plsc.subcore_barrier()

        e_base = wid * _NCH_T * _EC

        def ldidx(c, sbuf, dbuf):
            off = pl.multiple_of(e_base + c * _EC, _EC)
            pltpu.sync_copy(srcp.at[pl.ds(off, _EC)], sbuf)
            pltpu.sync_copy(dstp.at[pl.ds(off, _EC)], dbuf)

        def gath(buf, idx, sem):
            pltpu.async_copy(x.at[idx], buf, sem)

        def gath_wait(buf, idx, sem):
            pltpu.make_async_copy(x.at[idx], buf, sem).wait()

        def sadd(buf, idx, sem):
            pltpu.async_copy(buf, aggsp.at[idx], sem, add=True)

        def sadd_wait(buf, idx, sem):
            pltpu.make_async_copy(buf, aggsp.at[idx], sem).wait()

        ldidx(0, sbuf0, dbuf0)
        gath(rows0, sbuf0, gsem0)
        ldidx(1, sbuf1, dbuf1)
        gath(rows1, sbuf1, gsem1)

        def pipe(i, carry):
            c = 2 * i
            gath_wait(rows0, sbuf0, gsem0)
            sadd(rows0, dbuf0, asem0)
            gath_wait(rows1, sbuf1, gsem1)
            sadd(rows1, dbuf1, asem1)
            sadd_wait(rows0, dbuf0, asem0)
            ldidx(c + 2, sbuf0, dbuf0)
            gath(rows0, sbuf0, gsem0)
            sadd_wait(rows1, dbuf1, asem1)
            ldidx(c + 3, sbuf1, dbuf1)
            gath(rows1, sbuf1, gsem1)
            return carry

        lax.fori_loop(0, _NCH_T // 2 - 1, pipe, 0)
        gath_wait(rows0, sbuf0, gsem0)
        sadd(rows0, dbuf0, asem0)
        gath_wait(rows1, sbuf1, gsem1)
        sadd(rows1, dbuf1, asem1)
        sadd_wait(rows0, dbuf0, asem0)
        sadd_wait(rows1, dbuf1, asem1)
        plsc.subcore_barrier()

        for b in range(_ROWS_T // _EC):
            pltpu.sync_copy(aggsp.at[pl.ds(r0 + b * _EC, _EC)], rows0)

            @pl.when(core == 0)
            def _():
                pltpu.sync_copy(rows0, agg0.at[pl.ds(r0 + b * _EC, _EC)])

            @pl.when(core == 1)
            def _():
                pltpu.sync_copy(rows0, agg1.at[pl.ds(r0 + b * _EC, _EC)])

        if with_cnt:
            pltpu.sync_copy(z64.at[pl.ds(0, _EC)], rows1)
            for b in range(_ROWS_T // _EC):
                pltpu.sync_copy(rows1, aggsp.at[pl.ds(r0 + b * _EC, _EC)])
            pltpu.sync_copy(ones, rows0)
            plsc.subcore_barrier()

            ldidx(0, sbuf0, dbuf0)
            sadd(rows0, dbuf0, asem0)
            ldidx(1, sbuf1, dbuf1)
            sadd(rows0, dbuf1, asem1)

            def pipe_cnt(i, carry):
                c = 2 * i
                sadd_wait(rows0, dbuf0, asem0)
                ldidx(c + 2, sbuf0, dbuf0)
                sadd(rows0, dbuf0, asem0)
                sadd_wait(rows0, dbuf1, asem1)
                ldidx(c + 3, sbuf1, dbuf1)
                sadd(rows0, dbuf1, asem1)
                return carry

            lax.fori_loop(0, _NCH_T // 2 - 1, pipe_cnt, 0)
            sadd_wait(rows0, dbuf0, asem0)
            sadd_wait(rows0, dbuf1, asem1)
            plsc.subcore_barrier()

            for b in range(_ROWS_T // _EC):
                pltpu.sync_copy(aggsp.at[pl.ds(r0 + b * _EC, _EC)], rows0)

                @pl.when(core == 0)
                def _():
                    pltpu.sync_copy(rows0, cntw0.at[pl.ds(r0 + b * _EC, _EC)])

                @pl.when(core == 1)
                def _():
                    pltpu.sync_copy(rows0, cntw1.at[pl.ds(r0 + b * _EC, _EC)])

    return pl.kernel(body, out_type=tuple(outs), mesh=mesh,
                     scratch_types=scratch)


def _sc_agg(x, srcp, dstp, zeros64, with_cnt, ones=None):
    k = _make_sc_agg(with_cnt)
    if with_cnt:
        return k(x, srcp, dstp, zeros64, ones)
    return k(x, srcp, dstp, zeros64)


def _sort_pool_xla(x, batch):
    score = x[:, -1]
    order = jnp.lexsort((-score, batch))
    bs = batch[order]
    xs = x[order]
    counts = jnp.bincount(batch, length=_B)
    starts = jnp.cumsum(counts) - counts
    pos = jnp.arange(x.shape[0]) - starts[bs]
    mask = pos < _K
    safe = jnp.where(mask, pos, 0)
    vals = jnp.where(mask[:, None], xs, 0.0)
    return jnp.zeros((_B, _K, x.shape[1]), x.dtype).at[bs, safe].add(vals)


def kernel(x, edge_index, batch, k, Wl1, bl1, Wr1, Wl2, bl2, Wr2, Wl3, bl3,
           Wr3, Wc, bc, W1, b1, W2, b2):
    src, dst = edge_index[0], edge_index[1]
    srcp = jnp.concatenate([src, jnp.zeros((_EPAD,), jnp.int32)])
    dstp = jnp.concatenate([dst, _N + (jnp.arange(_EPAD, dtype=jnp.int32) % (_NP - _N))])
    zeros64 = jnp.zeros((_NP, _D), jnp.float32)
    ones = jnp.full((_EC, _D), 1.0, jnp.float32)

    agg0, agg1, cw0, cw1 = _sc_agg(x, srcp, dstp, zeros64, True, ones)
    h = _sage_dense(agg0, agg1, cw0, cw1, x, Wl1, bl1, Wr1)
    agg0, agg1 = _sc_agg(h, srcp, dstp, zeros64, False)
    h = _sage_dense(agg0, agg1, cw0, cw1, h, Wl2, bl2, Wr2)
    agg0, agg1 = _sc_agg(h, srcp, dstp, zeros64, False)
    h = _sage_dense(agg0, agg1, cw0, cw1, h, Wl3, bl3, Wr3)
    p = _sort_pool_xla(h, batch)
    return _head(p, Wc, bc, W1, b1, W2, b2)


# speedup vs baseline: 1.8817x; 1.8800x over previous
import functools

import jax
import jax.numpy as jnp
from jax import lax
from jax.experimental import pallas as pl
from jax.experimental.pallas import tpu as pltpu
from jax.experimental.pallas import tpu_sc as plsc

_N = 10000
_E = 320000
_D = 128
_B = 64
_K = 30
_CONV_OUT = 32
_CONV_W = 3
_NUM_CLASSES = 10

_ROWS_PER_BLK = 2000


def _sage_dense_body(mean_ref, x_ref, wl_ref, bl_ref, wr_ref, o_ref):
    m = mean_ref[...]
    x = x_ref[...]
    acc = jax.lax.dot_general(m, wl_ref[...], (((1,), (1,)), ((), ())),
                              preferred_element_type=jnp.float32)
    acc = acc + jax.lax.dot_general(x, wr_ref[...], (((1,), (1,)), ((), ())),
                                    preferred_element_type=jnp.float32)
    acc = acc + bl_ref[...]
    o_ref[...] = jnp.maximum(acc, 0.0)


def _sage_dense(mean, x, Wl, bl, Wr):
    nblk = _N // _ROWS_PER_BLK
    return pl.pallas_call(
        _sage_dense_body,
        grid=(nblk,),
        in_specs=[
            pl.BlockSpec((_ROWS_PER_BLK, _D), lambda i: (i, 0)),
            pl.BlockSpec((_ROWS_PER_BLK, _D), lambda i: (i, 0)),
            pl.BlockSpec((_D, _D), lambda i: (0, 0)),
            pl.BlockSpec((1, _D), lambda i: (0, 0)),
            pl.BlockSpec((_D, _D), lambda i: (0, 0)),
        ],
        out_specs=pl.BlockSpec((_ROWS_PER_BLK, _D), lambda i: (i, 0)),
        out_shape=jax.ShapeDtypeStruct((_N, _D), jnp.float32),
    )(mean, x, Wl, bl.reshape(1, _D), Wr)


def _head_body(p_ref, wc_ref, bc_ref, w1_ref, b1_ref, w2_ref, b2_ref, o_ref):
    p = p_ref[...].reshape(_B * _K, _D)
    q = jax.lax.dot_general(p, wc_ref[...], (((1,), (0,)), ((), ())),
                            preferred_element_type=jnp.float32)
    q = q.reshape(_B, _K, _CONV_W * _CONV_OUT)
    t_out = _K - _CONV_W + 1
    y = (q[:, 0:t_out, 0:_CONV_OUT]
         + q[:, 1:t_out + 1, _CONV_OUT:2 * _CONV_OUT]
         + q[:, 2:t_out + 2, 2 * _CONV_OUT:3 * _CONV_OUT])
    y = jnp.maximum(y + bc_ref[...], 0.0)
    z = y.reshape(_B, t_out * _CONV_OUT)
    z = jax.lax.dot_general(z, w1_ref[...], (((1,), (0,)), ((), ())),
                            preferred_element_type=jnp.float32)
    z = jnp.maximum(z + b1_ref[...], 0.0)
    logits = jax.lax.dot_general(z, w2_ref[...], (((1,), (0,)), ((), ())),
                                 preferred_element_type=jnp.float32)
    logits = logits + b2_ref[...]
    mx = jnp.max(logits, axis=-1, keepdims=True)
    s = logits - mx
    lse = jnp.log(jnp.sum(jnp.exp(s), axis=-1, keepdims=True))
    o_ref[...] = s - lse


def _head(p, Wc, bc, W1, b1, W2, b2):
    t_out = _K - _CONV_W + 1
    wc_all = jnp.concatenate([Wc[:, :, dt].T for dt in range(_CONV_W)], axis=1)
    w1p = W1.reshape(_D, _CONV_OUT, t_out).transpose(2, 1, 0).reshape(t_out * _CONV_OUT, _D)
    return pl.pallas_call(
        _head_body,
        out_shape=jax.ShapeDtypeStruct((_B, _NUM_CLASSES), jnp.float32),
    )(p, wc_all, bc.reshape(1, 1, _CONV_OUT), w1p, b1.reshape(1, _D),
      W2.T, b2.reshape(1, _NUM_CLASSES))


_EC = 128
_NCHUNK = _E // _EC
_NT = 16
_NW = 32
_NP = 10240
_ROWS_T = _NP // _NT


def _make_sc_agg(with_cnt):
    outs = [jax.ShapeDtypeStruct((_NP, _D), jnp.float32),
            jax.ShapeDtypeStruct((_NP, _D), jnp.float32)]
    scratch = [
        pltpu.VMEM((_EC,), jnp.int32),
        pltpu.VMEM((_EC,), jnp.int32),
        pltpu.VMEM((_EC, _D), jnp.float32),
        pltpu.VMEM_SHARED((_NP, _D), jnp.float32),
        pltpu.SemaphoreType.DMA,
    ]
    if with_cnt:
        outs += [jax.ShapeDtypeStruct((_NP, _D), jnp.float32),
                 jax.ShapeDtypeStruct((_NP, _D), jnp.float32)]
    mesh = plsc.VectorSubcoreMesh(core_axis_name="c", subcore_axis_name="s")

    def body(x, src, dst, z64, *refs):
        if with_cnt:
            (ones, agg0, agg1, cntw0, cntw1,
             sbuf, dbuf, rows, aggsp, sem) = refs
        else:
            (agg0, agg1, sbuf, dbuf, rows, aggsp, sem) = refs
        core = lax.axis_index("c")
        sid = lax.axis_index("s")
        wid = core * _NT + sid
        r0 = sid * _ROWS_T
        pltpu.sync_copy(z64.at[pl.ds(0, _EC)], rows)
        for b in range(_ROWS_T // _EC):
            pltpu.sync_copy(rows, aggsp.at[pl.ds(r0 + b * _EC, _EC)])
        plsc.subcore_barrier()

        rem = _NCHUNK % _NW
        nch = jnp.where(wid < rem, _NCHUNK // _NW + 1, _NCHUNK // _NW)

        def chunk(j, carry):
            cid = wid + _NW * j
            off = pl.multiple_of(cid * _EC, _EC)
            pltpu.sync_copy(src.at[pl.ds(off, _EC)], sbuf)
            pltpu.sync_copy(dst.at[pl.ds(off, _EC)], dbuf)
            pltpu.async_copy(x.at[sbuf], rows, sem).wait()
            pltpu.sync_copy(rows, aggsp.at[dbuf], add=True)
            return carry

        lax.fori_loop(0, nch, chunk, 0)
        plsc.subcore_barrier()

        for b in range(_ROWS_T // _EC):
            pltpu.sync_copy(aggsp.at[pl.ds(r0 + b * _EC, _EC)], rows)

            @pl.when(core == 0)
            def _():
                pltpu.sync_copy(rows, agg0.at[pl.ds(r0 + b * _EC, _EC)])

            @pl.when(core == 1)
            def _():
                pltpu.sync_copy(rows, agg1.at[pl.ds(r0 + b * _EC, _EC)])
        if with_cnt:
            pltpu.sync_copy(z64.at[pl.ds(0, _EC)], rows)
            for b in range(_ROWS_T // _EC):
                pltpu.sync_copy(rows, aggsp.at[pl.ds(r0 + b * _EC, _EC)])
            pltpu.sync_copy(ones, rows)
            plsc.subcore_barrier()

            def chunk_cnt(j, carry):
                cid = wid + _NW * j
                off = pl.multiple_of(cid * _EC, _EC)
                pltpu.sync_copy(dst.at[pl.ds(off, _EC)], dbuf)
                pltpu.sync_copy(rows, aggsp.at[dbuf], add=True)
                return carry

            lax.fori_loop(0, nch, chunk_cnt, 0)
            plsc.subcore_barrier()
            for b in range(_ROWS_T // _EC):
                pltpu.sync_copy(aggsp.at[pl.ds(r0 + b * _EC, _EC)], rows)

                @pl.when(core == 0)
                def _():
                    pltpu.sync_copy(rows, cntw0.at[pl.ds(r0 + b * _EC, _EC)])

                @pl.when(core == 1)
                def _():
                    pltpu.sync_copy(rows, cntw1.at[pl.ds(r0 + b * _EC, _EC)])

    return pl.kernel(body, out_type=tuple(outs), mesh=mesh,
                     scratch_types=scratch)


def _sc_agg(x, src, dst, zeros64, with_cnt, ones=None):
    k = _make_sc_agg(with_cnt)
    if with_cnt:
        return k(x, src, dst, zeros64, ones)
    return k(x, src, dst, zeros64)


def _sage_dense_body2(agg0_ref, agg1_ref, cw0_ref, cw1_ref, x_ref, wl_ref,
                      bl_ref, wr_ref, o_ref):
    cnt = cw0_ref[...][:, 0:1] + cw1_ref[...][:, 0:1]
    inv = 1.0 / jnp.maximum(cnt, 1.0)
    m = (agg0_ref[...] + agg1_ref[...]) * inv
    acc = jax.lax.dot_general(m, wl_ref[...], (((1,), (1,)), ((), ())),
                              preferred_element_type=jnp.float32)
    acc = acc + jax.lax.dot_general(x_ref[...], wr_ref[...],
                                    (((1,), (1,)), ((), ())),
                                    preferred_element_type=jnp.float32)
    acc = acc + bl_ref[...]
    o_ref[...] = jnp.maximum(acc, 0.0)


def _sage_dense2(agg0, agg1, cw0, cw1, x, Wl, bl, Wr):
    nblk = _N // _ROWS_PER_BLK
    return pl.pallas_call(
        _sage_dense_body2,
        grid=(nblk,),
        in_specs=[
            pl.BlockSpec((_ROWS_PER_BLK, _D), lambda i: (i, 0)),
            pl.BlockSpec((_ROWS_PER_BLK, _D), lambda i: (i, 0)),
            pl.BlockSpec((_ROWS_PER_BLK, _D), lambda i: (i, 0)),
            pl.BlockSpec((_ROWS_PER_BLK, _D), lambda i: (i, 0)),
            pl.BlockSpec((_ROWS_PER_BLK, _D), lambda i: (i, 0)),
            pl.BlockSpec((_D, _D), lambda i: (0, 0)),
            pl.BlockSpec((1, _D), lambda i: (0, 0)),
            pl.BlockSpec((_D, _D), lambda i: (0, 0)),
        ],
        out_specs=pl.BlockSpec((_ROWS_PER_BLK, _D), lambda i: (i, 0)),
        out_shape=jax.ShapeDtypeStruct((_N, _D), jnp.float32),
    )(agg0, agg1, cw0, cw1, x, Wl, bl.reshape(1, _D), Wr)


def _mean_agg_xla(x, src, dst):
    agg = jax.ops.segment_sum(x[src], dst, num_segments=_N)
    cnt = jax.ops.segment_sum(jnp.ones((src.shape[0],), x.dtype), dst, num_segments=_N)
    return agg / jnp.maximum(cnt, 1.0)[:, None]


def _sort_pool_xla(x, batch):
    score = x[:, -1]
    order = jnp.lexsort((-score, batch))
    bs = batch[order]
    xs = x[order]
    counts = jnp.bincount(batch, length=_B)
    starts = jnp.cumsum(counts) - counts
    pos = jnp.arange(x.shape[0]) - starts[bs]
    mask = pos < _K
    safe = jnp.where(mask, pos, 0)
    vals = jnp.where(mask[:, None], xs, 0.0)
    return jnp.zeros((_B, _K, x.shape[1]), x.dtype).at[bs, safe].add(vals)


def kernel(x, edge_index, batch, k, Wl1, bl1, Wr1, Wl2, bl2, Wr2, Wl3, bl3,
           Wr3, Wc, bc, W1, b1, W2, b2):
    src, dst = edge_index[0], edge_index[1]
    zeros64 = jnp.zeros((_NP, _D), jnp.float32)
    ones = jnp.full((_EC, _D), 1.0, jnp.float32)

    agg0, agg1, cw0, cw1 = _sc_agg(x, src, dst, zeros64, True, ones)
    h = _sage_dense2(agg0, agg1, cw0, cw1, x, Wl1, bl1, Wr1)
    agg0, agg1 = _sc_agg(h, src, dst, zeros64, False)
    h = _sage_dense2(agg0, agg1, cw0, cw1, h, Wl2, bl2, Wr2)
    agg0, agg1 = _sc_agg(h, src, dst, zeros64, False)
    h = _sage_dense2(agg0, agg1, cw0, cw1, h, Wl3, bl3, Wr3)
    p = _sort_pool_xla(h, batch)
    return _head(p, Wc, bc, W1, b1, W2, b2)


# speedup vs baseline: 2.3289x; 1.2377x over previous
import functools

import jax
import jax.numpy as jnp
from jax import lax
from jax.experimental import pallas as pl
from jax.experimental.pallas import tpu as pltpu
from jax.experimental.pallas import tpu_sc as plsc

_N = 10000
_E = 320000
_D = 128
_B = 64
_K = 30
_CONV_OUT = 32
_CONV_W = 3
_NUM_CLASSES = 10

_ROWS_PER_BLK = 2000


def _sage_dense_body(mean_ref, x_ref, wl_ref, bl_ref, wr_ref, o_ref):
    m = mean_ref[...]
    x = x_ref[...]
    acc = jax.lax.dot_general(m, wl_ref[...], (((1,), (1,)), ((), ())),
                              preferred_element_type=jnp.float32)
    acc = acc + jax.lax.dot_general(x, wr_ref[...], (((1,), (1,)), ((), ())),
                                    preferred_element_type=jnp.float32)
    acc = acc + bl_ref[...]
    o_ref[...] = jnp.maximum(acc, 0.0)


def _sage_dense(mean, x, Wl, bl, Wr):
    nblk = _N // _ROWS_PER_BLK
    return pl.pallas_call(
        _sage_dense_body,
        grid=(nblk,),
        in_specs=[
            pl.BlockSpec((_ROWS_PER_BLK, _D), lambda i: (i, 0)),
            pl.BlockSpec((_ROWS_PER_BLK, _D), lambda i: (i, 0)),
            pl.BlockSpec((_D, _D), lambda i: (0, 0)),
            pl.BlockSpec((1, _D), lambda i: (0, 0)),
            pl.BlockSpec((_D, _D), lambda i: (0, 0)),
        ],
        out_specs=pl.BlockSpec((_ROWS_PER_BLK, _D), lambda i: (i, 0)),
        out_shape=jax.ShapeDtypeStruct((_N, _D), jnp.float32),
    )(mean, x, Wl, bl.reshape(1, _D), Wr)


def _head_body(p_ref, wc_ref, bc_ref, w1_ref, b1_ref, w2_ref, b2_ref, o_ref):
    p = p_ref[...].reshape(_B * _K, _D)
    q = jax.lax.dot_general(p, wc_ref[...], (((1,), (0,)), ((), ())),
                            preferred_element_type=jnp.float32)
    q = q.reshape(_B, _K, _CONV_W * _CONV_OUT)
    t_out = _K - _CONV_W + 1
    y = (q[:, 0:t_out, 0:_CONV_OUT]
         + q[:, 1:t_out + 1, _CONV_OUT:2 * _CONV_OUT]
         + q[:, 2:t_out + 2, 2 * _CONV_OUT:3 * _CONV_OUT])
    y = jnp.maximum(y + bc_ref[...], 0.0)
    z = y.reshape(_B, t_out * _CONV_OUT)
    z = jax.lax.dot_general(z, w1_ref[...], (((1,), (0,)), ((), ())),
                            preferred_element_type=jnp.float32)
    z = jnp.maximum(z + b1_ref[...], 0.0)
    logits = jax.lax.dot_general(z, w2_ref[...], (((1,), (0,)), ((), ())),
                                 preferred_element_type=jnp.float32)
    logits = logits + b2_ref[...]
    mx = jnp.max(logits, axis=-1, keepdims=True)
    s = logits - mx
    lse = jnp.log(jnp.sum(jnp.exp(s), axis=-1, keepdims=True))
    o_ref[...] = s - lse


def _head(p, Wc, bc, W1, b1, W2, b2):
    t_out = _K - _CONV_W + 1
    wc_all = jnp.concatenate([Wc[:, :, dt].T for dt in range(_CONV_W)], axis=1)
    w1p = W1.reshape(_D, _CONV_OUT, t_out).transpose(2, 1, 0).reshape(t_out * _CONV_OUT, _D)
    return pl.pallas_call(
        _head_body,
        out_shape=jax.ShapeDtypeStruct((_B, _NUM_CLASSES), jnp.float32),
    )(p, wc_all, bc.reshape(1, 1, _CONV_OUT), w1p, b1.reshape(1, _D),
      W2.T, b2.reshape(1, _NUM_CLASSES))


_EC = 128
_NCHUNK = _E // _EC
_NT = 16
_NW = 32
_NP = 10240
_ROWS_T = _NP // _NT


def _make_sc_agg(with_cnt):
    outs = [jax.ShapeDtypeStruct((_NP, _D), jnp.float32),
            jax.ShapeDtypeStruct((_NP, _D), jnp.float32)]
    scratch = [
        pltpu.VMEM((2 * _EC,), jnp.int32),
        pltpu.VMEM((_EC,), jnp.int32),
        pltpu.VMEM((_EC, _D), jnp.float32),
        pltpu.VMEM_SHARED((_NP, _D), jnp.float32),
        pltpu.SemaphoreType.DMA,
        pltpu.SemaphoreType.DMA,
    ]
    if with_cnt:
        outs += [jax.ShapeDtypeStruct((_NP, _D), jnp.float32),
                 jax.ShapeDtypeStruct((_NP, _D), jnp.float32)]
    mesh = plsc.VectorSubcoreMesh(core_axis_name="c", subcore_axis_name="s")

    def body(x, ecomb, dst, z64, *refs):
        if with_cnt:
            (ones, agg0, agg1, cntw0, cntw1,
             ebuf, dbuf, rows, aggsp, sem, isem) = refs
        else:
            (agg0, agg1, ebuf, dbuf, rows, aggsp, sem, isem) = refs
        core = lax.axis_index("c")
        sid = lax.axis_index("s")
        wid = core * _NT + sid
        r0 = sid * _ROWS_T
        pltpu.sync_copy(z64.at[pl.ds(0, _EC)], rows)
        for b in range(_ROWS_T // _EC):
            pltpu.sync_copy(rows, aggsp.at[pl.ds(r0 + b * _EC, _EC)])
        plsc.subcore_barrier()

        rem = _NCHUNK % _NW
        nch = jnp.where(wid < rem, _NCHUNK // _NW + 1, _NCHUNK // _NW)

        def eoff(j):
            cid = jnp.minimum(wid + _NW * j, _NCHUNK - 1)
            return pl.multiple_of(cid * (2 * _EC), 2 * _EC)

        pltpu.async_copy(ecomb.at[pl.ds(eoff(0), 2 * _EC)], ebuf, isem)

        def chunk(j, carry):
            pltpu.make_async_copy(
                ecomb.at[pl.ds(eoff(j), 2 * _EC)], ebuf, isem).wait()
            pltpu.async_copy(x.at[ebuf.at[pl.ds(0, _EC)]], rows, sem).wait()
            for u in range(_EC // 16):
                dbuf[pl.ds(u * 16, 16)] = ebuf[pl.ds(_EC + u * 16, 16)]
            pltpu.async_copy(ecomb.at[pl.ds(eoff(j + 1), 2 * _EC)], ebuf, isem)
            pltpu.sync_copy(rows, aggsp.at[dbuf], add=True)
            return carry

        lax.fori_loop(0, nch, chunk, 0)
        pltpu.make_async_copy(
            ecomb.at[pl.ds(eoff(0), 2 * _EC)], ebuf, isem).wait()
        plsc.subcore_barrier()

        for b in range(_ROWS_T // _EC):
            pltpu.sync_copy(aggsp.at[pl.ds(r0 + b * _EC, _EC)], rows)

            @pl.when(core == 0)
            def _():
                pltpu.sync_copy(rows, agg0.at[pl.ds(r0 + b * _EC, _EC)])

            @pl.when(core == 1)
            def _():
                pltpu.sync_copy(rows, agg1.at[pl.ds(r0 + b * _EC, _EC)])
        if with_cnt:
            pltpu.sync_copy(z64.at[pl.ds(0, _EC)], rows)
            for b in range(_ROWS_T // _EC):
                pltpu.sync_copy(rows, aggsp.at[pl.ds(r0 + b * _EC, _EC)])
            pltpu.sync_copy(ones, rows)
            plsc.subcore_barrier()

            def chunk_cnt(j, carry):
                cid = wid + _NW * j
                off = pl.multiple_of(cid * _EC, _EC)
                pltpu.sync_copy(dst.at[pl.ds(off, _EC)], dbuf)
                pltpu.sync_copy(rows, aggsp.at[dbuf], add=True)
                return carry

            lax.fori_loop(0, nch, chunk_cnt, 0)
            plsc.subcore_barrier()
            for b in range(_ROWS_T // _EC):
                pltpu.sync_copy(aggsp.at[pl.ds(r0 + b * _EC, _EC)], rows)

                @pl.when(core == 0)
                def _():
                    pltpu.sync_copy(rows, cntw0.at[pl.ds(r0 + b * _EC, _EC)])

                @pl.when(core == 1)
                def _():
                    pltpu.sync_copy(rows, cntw1.at[pl.ds(r0 + b * _EC, _EC)])

    return pl.kernel(body, out_type=tuple(outs), mesh=mesh,
                     scratch_types=scratch)


def _sc_agg(x, ecomb, dst, zeros64, with_cnt, ones=None):
    k = _make_sc_agg(with_cnt)
    if with_cnt:
        return k(x, ecomb, dst, zeros64, ones)
    return k(x, ecomb, dst, zeros64)


def _sage_dense_body2(agg0_ref, agg1_ref, cw0_ref, cw1_ref, x_ref, wl_ref,
                      bl_ref, wr_ref, o_ref):
    cnt = cw0_ref[...][:, 0:1] + cw1_ref[...][:, 0:1]
    inv = 1.0 / jnp.maximum(cnt, 1.0)
    m = (agg0_ref[...] + agg1_ref[...]) * inv
    acc = jax.lax.dot_general(m, wl_ref[...], (((1,), (1,)), ((), ())),
                              preferred_element_type=jnp.float32)
    acc = acc + jax.lax.dot_general(x_ref[...], wr_ref[...],
                                    (((1,), (1,)), ((), ())),
                                    preferred_element_type=jnp.float32)
    acc = acc + bl_ref[...]
    o_ref[...] = jnp.maximum(acc, 0.0)


def _sage_dense2(agg0, agg1, cw0, cw1, x, Wl, bl, Wr):
    nblk = _N // _ROWS_PER_BLK
    return pl.pallas_call(
        _sage_dense_body2,
        grid=(nblk,),
        in_specs=[
            pl.BlockSpec((_ROWS_PER_BLK, _D), lambda i: (i, 0)),
            pl.BlockSpec((_ROWS_PER_BLK, _D), lambda i: (i, 0)),
            pl.BlockSpec((_ROWS_PER_BLK, _D), lambda i: (i, 0)),
            pl.BlockSpec((_ROWS_PER_BLK, _D), lambda i: (i, 0)),
            pl.BlockSpec((_ROWS_PER_BLK, _D), lambda i: (i, 0)),
            pl.BlockSpec((_D, _D), lambda i: (0, 0)),
            pl.BlockSpec((1, _D), lambda i: (0, 0)),
            pl.BlockSpec((_D, _D), lambda i: (0, 0)),
        ],
        out_specs=pl.BlockSpec((_ROWS_PER_BLK, _D), lambda i: (i, 0)),
        out_shape=jax.ShapeDtypeStruct((_N, _D), jnp.float32),
    )(agg0, agg1, cw0, cw1, x, Wl, bl.reshape(1, _D), Wr)


def _mean_agg_xla(x, src, dst):
    agg = jax.ops.segment_sum(x[src], dst, num_segments=_N)
    cnt = jax.ops.segment_sum(jnp.ones((src.shape[0],), x.dtype), dst, num_segments=_N)
    return agg / jnp.maximum(cnt, 1.0)[:, None]


def _sort_pool_xla(x, batch):
    score = x[:, -1]
    order = jnp.lexsort((-score, batch))
    bs = batch[order]
    xs = x[order]
    counts = jnp.bincount(batch, length=_B)
    starts = jnp.cumsum(counts) - counts
    pos = jnp.arange(x.shape[0]) - starts[bs]
    mask = pos < _K
    safe = jnp.where(mask, pos, 0)
    vals = jnp.where(mask[:, None], xs, 0.0)
    return jnp.zeros((_B, _K, x.shape[1]), x.dtype).at[bs, safe].add(vals)


def kernel(x, edge_index, batch, k, Wl1, bl1, Wr1, Wl2, bl2, Wr2, Wl3, bl3,
           Wr3, Wc, bc, W1, b1, W2, b2):
    src, dst = edge_index[0], edge_index[1]
    ecomb = jnp.concatenate(
        [src.reshape(_NCHUNK, _EC), dst.reshape(_NCHUNK, _EC)],
        axis=1).reshape(-1)
    zeros64 = jnp.zeros((_NP, _D), jnp.float32)
    ones = jnp.full((_EC, _D), 1.0, jnp.float32)

    agg0, agg1, cw0, cw1 = _sc_agg(x, ecomb, dst, zeros64, True, ones)
    h = _sage_dense2(agg0, agg1, cw0, cw1, x, Wl1, bl1, Wr1)
    agg0, agg1 = _sc_agg(h, ecomb, dst, zeros64, False)
    h = _sage_dense2(agg0, agg1, cw0, cw1, h, Wl2, bl2, Wr2)
    agg0, agg1 = _sc_agg(h, ecomb, dst, zeros64, False)
    h = _sage_dense2(agg0, agg1, cw0, cw1, h, Wl3, bl3, Wr3)
    p = _sort_pool_xla(h, batch)
    return _head(p, Wc, bc, W1, b1, W2, b2)


# speedup vs baseline: 2.4153x; 1.0371x over previous
import functools

import jax
import jax.numpy as jnp
from jax import lax
from jax.experimental import pallas as pl
from jax.experimental.pallas import tpu as pltpu
from jax.experimental.pallas import tpu_sc as plsc

_N = 10000
_E = 320000
_D = 128
_B = 64
_K = 30
_CONV_OUT = 32
_CONV_W = 3
_NUM_CLASSES = 10

_ROWS_PER_BLK = 2000


def _sage_dense_body(mean_ref, x_ref, wl_ref, bl_ref, wr_ref, o_ref):
    m = mean_ref[...]
    x = x_ref[...]
    acc = jax.lax.dot_general(m, wl_ref[...], (((1,), (1,)), ((), ())),
                              preferred_element_type=jnp.float32)
    acc = acc + jax.lax.dot_general(x, wr_ref[...], (((1,), (1,)), ((), ())),
                                    preferred_element_type=jnp.float32)
    acc = acc + bl_ref[...]
    o_ref[...] = jnp.maximum(acc, 0.0)


def _sage_dense(mean, x, Wl, bl, Wr):
    nblk = _N // _ROWS_PER_BLK
    return pl.pallas_call(
        _sage_dense_body,
        grid=(nblk,),
        in_specs=[
            pl.BlockSpec((_ROWS_PER_BLK, _D), lambda i: (i, 0)),
            pl.BlockSpec((_ROWS_PER_BLK, _D), lambda i: (i, 0)),
            pl.BlockSpec((_D, _D), lambda i: (0, 0)),
            pl.BlockSpec((1, _D), lambda i: (0, 0)),
            pl.BlockSpec((_D, _D), lambda i: (0, 0)),
        ],
        out_specs=pl.BlockSpec((_ROWS_PER_BLK, _D), lambda i: (i, 0)),
        out_shape=jax.ShapeDtypeStruct((_N, _D), jnp.float32),
    )(mean, x, Wl, bl.reshape(1, _D), Wr)


def _head_body(p_ref, wc_ref, bc_ref, w1_ref, b1_ref, w2_ref, b2_ref, o_ref):
    p = p_ref[...].reshape(_B * _K, _D)
    q = jax.lax.dot_general(p, wc_ref[...], (((1,), (0,)), ((), ())),
                            preferred_element_type=jnp.float32)
    q = q.reshape(_B, _K, _CONV_W * _CONV_OUT)
    t_out = _K - _CONV_W + 1
    y = (q[:, 0:t_out, 0:_CONV_OUT]
         + q[:, 1:t_out + 1, _CONV_OUT:2 * _CONV_OUT]
         + q[:, 2:t_out + 2, 2 * _CONV_OUT:3 * _CONV_OUT])
    y = jnp.maximum(y + bc_ref[...], 0.0)
    z = y.reshape(_B, t_out * _CONV_OUT)
    z = jax.lax.dot_general(z, w1_ref[...], (((1,), (0,)), ((), ())),
                            preferred_element_type=jnp.float32)
    z = jnp.maximum(z + b1_ref[...], 0.0)
    logits = jax.lax.dot_general(z, w2_ref[...], (((1,), (0,)), ((), ())),
                                 preferred_element_type=jnp.float32)
    logits = logits + b2_ref[...]
    mx = jnp.max(logits, axis=-1, keepdims=True)
    s = logits - mx
    lse = jnp.log(jnp.sum(jnp.exp(s), axis=-1, keepdims=True))
    o_ref[...] = s - lse


def _head(p, Wc, bc, W1, b1, W2, b2):
    t_out = _K - _CONV_W + 1
    wc_all = jnp.concatenate([Wc[:, :, dt].T for dt in range(_CONV_W)], axis=1)
    w1p = W1.reshape(_D, _CONV_OUT, t_out).transpose(2, 1, 0).reshape(t_out * _CONV_OUT, _D)
    return pl.pallas_call(
        _head_body,
        out_shape=jax.ShapeDtypeStruct((_B, _NUM_CLASSES), jnp.float32),
    )(p, wc_all, bc.reshape(1, 1, _CONV_OUT), w1p, b1.reshape(1, _D),
      W2.T, b2.reshape(1, _NUM_CLASSES))


_EC = 128
_NCHUNK = _E // _EC
_NT = 16
_NW = 32
_NP = 10240
_ROWS_T = _NP // _NT


def _make_sc_agg(with_cnt):
    outs = [jax.ShapeDtypeStruct((_NP, _D), jnp.float32),
            jax.ShapeDtypeStruct((_NP, _D), jnp.float32)]
    scratch = [
        pltpu.VMEM((2 * _EC,), jnp.int32),
        pltpu.VMEM((_EC,), jnp.int32),
        pltpu.VMEM((_EC, _D), jnp.float32),
        pltpu.VMEM_SHARED((_NP, _D), jnp.float32),
        pltpu.SemaphoreType.DMA,
        pltpu.SemaphoreType.DMA,
    ]
    if with_cnt:
        outs += [jax.ShapeDtypeStruct((_NP, _D), jnp.float32),
                 jax.ShapeDtypeStruct((_NP, _D), jnp.float32)]
    mesh = plsc.VectorSubcoreMesh(core_axis_name="c", subcore_axis_name="s")

    def body(x, ecomb, z64, *refs):
        if with_cnt:
            (ones, agg0, agg1, cntw0, cntw1,
             ebuf, dbuf, rows, aggsp, sem, isem) = refs
        else:
            (agg0, agg1, ebuf, dbuf, rows, aggsp, sem, isem) = refs
        core = lax.axis_index("c")
        sid = lax.axis_index("s")
        wid = core * _NT + sid
        r0 = sid * _ROWS_T
        pltpu.sync_copy(z64.at[pl.ds(0, _EC)], rows)
        for b in range(_ROWS_T // _EC):
            pltpu.sync_copy(rows, aggsp.at[pl.ds(r0 + b * _EC, _EC)])
        plsc.subcore_barrier()

        rem = _NCHUNK % _NW
        nch = jnp.where(wid < rem, _NCHUNK // _NW + 1, _NCHUNK // _NW)

        def eoff(j):
            cid = jnp.minimum(wid + _NW * j, _NCHUNK - 1)
            return pl.multiple_of(cid * (2 * _EC), 2 * _EC)

        pltpu.async_copy(ecomb.at[pl.ds(eoff(0), 2 * _EC)], ebuf, isem)

        def chunk(j, carry):
            pltpu.make_async_copy(
                ecomb.at[pl.ds(eoff(j), 2 * _EC)], ebuf, isem).wait()
            pltpu.async_copy(x.at[ebuf.at[pl.ds(0, _EC)]], rows, sem).wait()
            for u in range(_EC // 16):
                dbuf[pl.ds(u * 16, 16)] = ebuf[pl.ds(_EC + u * 16, 16)]
            pltpu.async_copy(ecomb.at[pl.ds(eoff(j + 1), 2 * _EC)], ebuf, isem)
            pltpu.sync_copy(rows, aggsp.at[dbuf], add=True)
            return carry

        lax.fori_loop(0, nch, chunk, 0)
        pltpu.make_async_copy(
            ecomb.at[pl.ds(eoff(0), 2 * _EC)], ebuf, isem).wait()
        plsc.subcore_barrier()

        for b in range(_ROWS_T // _EC):
            pltpu.sync_copy(aggsp.at[pl.ds(r0 + b * _EC, _EC)], rows)

            @pl.when(core == 0)
            def _():
                pltpu.sync_copy(rows, agg0.at[pl.ds(r0 + b * _EC, _EC)])

            @pl.when(core == 1)
            def _():
                pltpu.sync_copy(rows, agg1.at[pl.ds(r0 + b * _EC, _EC)])
        if with_cnt:
            pltpu.sync_copy(z64.at[pl.ds(0, _EC)], rows)
            for b in range(_ROWS_T // _EC):
                pltpu.sync_copy(rows, aggsp.at[pl.ds(r0 + b * _EC, _EC)])
            pltpu.sync_copy(ones, rows)
            plsc.subcore_barrier()

            pltpu.async_copy(ecomb.at[pl.ds(eoff(0), 2 * _EC)], ebuf, isem)

            def chunk_cnt(j, carry):
                pltpu.make_async_copy(
                    ecomb.at[pl.ds(eoff(j), 2 * _EC)], ebuf, isem).wait()
                for u in range(_EC // 16):
                    dbuf[pl.ds(u * 16, 16)] = ebuf[pl.ds(_EC + u * 16, 16)]
                pltpu.async_copy(
                    ecomb.at[pl.ds(eoff(j + 1), 2 * _EC)], ebuf, isem)
                pltpu.sync_copy(rows, aggsp.at[dbuf], add=True)
                return carry

            lax.fori_loop(0, nch, chunk_cnt, 0)
            pltpu.make_async_copy(
                ecomb.at[pl.ds(eoff(0), 2 * _EC)], ebuf, isem).wait()
            plsc.subcore_barrier()
            for b in range(_ROWS_T // _EC):
                pltpu.sync_copy(aggsp.at[pl.ds(r0 + b * _EC, _EC)], rows)

                @pl.when(core == 0)
                def _():
                    pltpu.sync_copy(rows, cntw0.at[pl.ds(r0 + b * _EC, _EC)])

                @pl.when(core == 1)
                def _():
                    pltpu.sync_copy(rows, cntw1.at[pl.ds(r0 + b * _EC, _EC)])

    return pl.kernel(body, out_type=tuple(outs), mesh=mesh,
                     scratch_types=scratch)


def _sc_agg(x, ecomb, zeros64, with_cnt, ones=None):
    k = _make_sc_agg(with_cnt)
    if with_cnt:
        return k(x, ecomb, zeros64, ones)
    return k(x, ecomb, zeros64)


def _sage_dense_body2(agg0_ref, agg1_ref, cw0_ref, cw1_ref, x_ref, wl_ref,
                      bl_ref, wr_ref, o_ref):
    cnt = cw0_ref[...][:, 0:1] + cw1_ref[...][:, 0:1]
    inv = 1.0 / jnp.maximum(cnt, 1.0)
    m = (agg0_ref[...] + agg1_ref[...]) * inv
    acc = jax.lax.dot_general(m, wl_ref[...], (((1,), (1,)), ((), ())),
                              preferred_element_type=jnp.float32)
    acc = acc + jax.lax.dot_general(x_ref[...], wr_ref[...],
                                    (((1,), (1,)), ((), ())),
                                    preferred_element_type=jnp.float32)
    acc = acc + bl_ref[...]
    o_ref[...] = jnp.maximum(acc, 0.0)


def _sage_dense2(agg0, agg1, cw0, cw1, x, Wl, bl, Wr):
    nblk = _N // _ROWS_PER_BLK
    return pl.pallas_call(
        _sage_dense_body2,
        grid=(nblk,),
        in_specs=[
            pl.BlockSpec((_ROWS_PER_BLK, _D), lambda i: (i, 0)),
            pl.BlockSpec((_ROWS_PER_BLK, _D), lambda i: (i, 0)),
            pl.BlockSpec((_ROWS_PER_BLK, _D), lambda i: (i, 0)),
            pl.BlockSpec((_ROWS_PER_BLK, _D), lambda i: (i, 0)),
            pl.BlockSpec((_ROWS_PER_BLK, _D), lambda i: (i, 0)),
            pl.BlockSpec((_D, _D), lambda i: (0, 0)),
            pl.BlockSpec((1, _D), lambda i: (0, 0)),
            pl.BlockSpec((_D, _D), lambda i: (0, 0)),
        ],
        out_specs=pl.BlockSpec((_ROWS_PER_BLK, _D), lambda i: (i, 0)),
        out_shape=jax.ShapeDtypeStruct((_N, _D), jnp.float32),
    )(agg0, agg1, cw0, cw1, x, Wl, bl.reshape(1, _D), Wr)


def _mean_agg_xla(x, src, dst):
    agg = jax.ops.segment_sum(x[src], dst, num_segments=_N)
    cnt = jax.ops.segment_sum(jnp.ones((src.shape[0],), x.dtype), dst, num_segments=_N)
    return agg / jnp.maximum(cnt, 1.0)[:, None]


def _sort_pool_xla(x, batch):
    score = x[:, -1]
    order = jnp.lexsort((-score, batch))
    bs = batch[order]
    xs = x[order]
    counts = jnp.bincount(batch, length=_B)
    starts = jnp.cumsum(counts) - counts
    pos = jnp.arange(x.shape[0]) - starts[bs]
    mask = pos < _K
    safe = jnp.where(mask, pos, 0)
    vals = jnp.where(mask[:, None], xs, 0.0)
    return jnp.zeros((_B, _K, x.shape[1]), x.dtype).at[bs, safe].add(vals)


def kernel(x, edge_index, batch, k, Wl1, bl1, Wr1, Wl2, bl2, Wr2, Wl3, bl3,
           Wr3, Wc, bc, W1, b1, W2, b2):
    src, dst = edge_index[0], edge_index[1]
    ecomb = jnp.concatenate(
        [src.reshape(_NCHUNK, _EC), dst.reshape(_NCHUNK, _EC)],
        axis=1).reshape(-1)
    zeros64 = jnp.zeros((_NP, _D), jnp.float32)
    ones = jnp.full((_EC, _D), 1.0, jnp.float32)

    agg0, agg1, cw0, cw1 = _sc_agg(x, ecomb, zeros64, True, ones)
    h = _sage_dense2(agg0, agg1, cw0, cw1, x, Wl1, bl1, Wr1)
    agg0, agg1 = _sc_agg(h, ecomb, zeros64, False)
    h = _sage_dense2(agg0, agg1, cw0, cw1, h, Wl2, bl2, Wr2)
    agg0, agg1 = _sc_agg(h, ecomb, zeros64, False)
    h = _sage_dense2(agg0, agg1, cw0, cw1, h, Wl3, bl3, Wr3)
    p = _sort_pool_xla(h, batch)
    return _head(p, Wc, bc, W1, b1, W2, b2)
